# baseline (speedup 1.0000x reference)
import functools

import jax
import jax.numpy as jnp
from jax import lax
from jax.experimental import pallas as pl
from jax.experimental.pallas import tpu as pltpu
from jax.experimental.pallas import tpu_sc as plsc

BUFFER_SIZE = 10000
N_CLASSES = 100
BATCH = 4096
ROW = 3 * 32 * 32

R1 = BATCH * ROW
R2 = (BUFFER_SIZE - BATCH) * ROW
TOT = BUFFER_SIZE * ROW

NTILES = 32
S1 = R1 // NTILES
S2 = R2 // NTILES

CHUNK = 49_152
N1 = S1 // CHUNK
N2 = S2 // CHUNK


def _body(x_h, y_h, t_h, bx_h, by_h, bt_h, obx_h, oby_h, obt_h,
          buf0, buf1, tfill, tailb, tvec, sem0, sem1):
    wid = lax.axis_index("s") * 2 + lax.axis_index("c")

    bufs = (buf0, buf1)
    semb = (sem0, sem1)

    def copy_span(src_h, src_off, dst_off, sizes):
        n = len(sizes)
        offs = [0]
        for s in sizes[:-1]:
            offs.append(offs[-1] + s)
        in_d = [None] * n
        out_d = [None] * n
        for i in range(n):
            b = bufs[i % 2]
            if i >= 2:
                out_d[i - 2].wait()
            in_d[i] = pltpu.async_copy(
                src_h.at[pl.ds(src_off + offs[i], sizes[i])],
                b.at[pl.ds(0, sizes[i])], semb[i % 2])
            in_d[i].wait()
            out_d[i] = pltpu.async_copy(
                b.at[pl.ds(0, sizes[i])],
                obx_h.at[pl.ds(dst_off + offs[i], sizes[i])], semb[i % 2])
        for i in range(max(0, n - 2), n):
            out_d[i].wait()

    copy_span(x_h, wid * S1, wid * S1, [CHUNK] * N1)
    base2 = R1 + wid * S2
    copy_span(bx_h, base2, base2, [CHUNK] * 11 + [S2 - 11 * CHUNK])

    TAIL = BUFFER_SIZE - BATCH

    @pl.when(wid == 30)
    def _():
        pltpu.sync_copy(y_h, tfill)
        pltpu.sync_copy(tfill, oby_h.at[pl.ds(0, BATCH)])
        pltpu.sync_copy(by_h.at[pl.ds(BATCH, TAIL)], tailb)
        pltpu.sync_copy(tailb, oby_h.at[pl.ds(BATCH, TAIL)])

    @pl.when(wid == 31)
    def _():
        pltpu.sync_copy(t_h, tvec)
        tv = tvec[...]
        for i in range(BATCH // 16):
            tfill[pl.ds(i * 16, 16)] = tv
        pltpu.sync_copy(tfill, obt_h.at[pl.ds(0, BATCH)])
        pltpu.sync_copy(bt_h.at[pl.ds(BATCH, TAIL)], tailb)
        pltpu.sync_copy(tailb, obt_h.at[pl.ds(BATCH, TAIL)])


@jax.jit
def _er_update(x, y, t16, bx, by, bt):
    xf = x.reshape(R1)
    bxf = bx.reshape(TOT)
    mesh = plsc.VectorSubcoreMesh(core_axis_name="c", subcore_axis_name="s")
    run = pl.kernel(
        _body,
        out_type=(
            jax.ShapeDtypeStruct((TOT,), jnp.float32),
            jax.ShapeDtypeStruct((BUFFER_SIZE,), jnp.int32),
            jax.ShapeDtypeStruct((BUFFER_SIZE,), jnp.int32),
        ),
        mesh=mesh,
        scratch_types=[
            pltpu.VMEM((CHUNK,), jnp.float32),
            pltpu.VMEM((CHUNK,), jnp.float32),
            pltpu.VMEM((BATCH,), jnp.int32),
            pltpu.VMEM((BUFFER_SIZE - BATCH,), jnp.int32),
            pltpu.VMEM((16,), jnp.int32),
            pltpu.SemaphoreType.DMA,
            pltpu.SemaphoreType.DMA,
        ],
    )
    obx, oby, obt = run(xf, y, t16, bxf, by, bt)
    return obx.reshape(bx.shape), oby, obt


def kernel(x, y, task_id, bx, by, bt):
    t16 = jnp.full((16,), task_id, dtype=jnp.int32)
    return _er_update(x, y, t16, bx, by, bt)


# speedup vs baseline: 1.0056x; 1.0056x over previous
import functools

import jax
import jax.numpy as jnp
from jax import lax
from jax.experimental import pallas as pl
from jax.experimental.pallas import tpu as pltpu
from jax.experimental.pallas import tpu_sc as plsc

BUFFER_SIZE = 10000
N_CLASSES = 100
BATCH = 4096
ROW = 3 * 32 * 32

R1 = BATCH * ROW
R2 = (BUFFER_SIZE - BATCH) * ROW
TOT = BUFFER_SIZE * ROW

NTILES = 32
S1 = R1 // NTILES
S2 = R2 // NTILES

CHUNK = 49_152
N1 = S1 // CHUNK
N2 = S2 // CHUNK


def _body(x_h, y_h, t_h, bx_h, by_h, bt_h, obx_h, oby_h, obt_h,
          spbuf, tfill, tailb, tvec, sem0, sem1):
    cid = lax.axis_index("c")
    sid = lax.axis_index("s")
    wid = sid * 2 + cid

    semb = (sem0, sem1)

    def copy_span(src_h, src_off, dst_off, sizes):
        n = len(sizes)
        offs = [0]
        for s in sizes[:-1]:
            offs.append(offs[-1] + s)
        in_d = [None] * n
        out_d = [None] * n
        for i in range(n):
            b = spbuf.at[sid, i % 2]
            if i >= 2:
                out_d[i - 2].wait()
            in_d[i] = pltpu.async_copy(
                src_h.at[pl.ds(src_off + offs[i], sizes[i])],
                b.at[pl.ds(0, sizes[i])], semb[i % 2])
            in_d[i].wait()
            out_d[i] = pltpu.async_copy(
                b.at[pl.ds(0, sizes[i])],
                obx_h.at[pl.ds(dst_off + offs[i], sizes[i])], semb[i % 2])
        for i in range(max(0, n - 2), n):
            out_d[i].wait()

    copy_span(x_h, wid * S1, wid * S1, [CHUNK] * N1)
    base2 = R1 + wid * S2
    copy_span(bx_h, base2, base2, [CHUNK] * 11 + [S2 - 11 * CHUNK])

    TAIL = BUFFER_SIZE - BATCH

    @pl.when(wid == 30)
    def _():
        pltpu.sync_copy(y_h, tfill)
        pltpu.sync_copy(tfill, oby_h.at[pl.ds(0, BATCH)])
        pltpu.sync_copy(by_h.at[pl.ds(BATCH, TAIL)], tailb)
        pltpu.sync_copy(tailb, oby_h.at[pl.ds(BATCH, TAIL)])

    @pl.when(wid == 31)
    def _():
        pltpu.sync_copy(t_h, tvec)
        tv = tvec[...]
        for i in range(BATCH // 16):
            tfill[pl.ds(i * 16, 16)] = tv
        pltpu.sync_copy(tfill, obt_h.at[pl.ds(0, BATCH)])
        pltpu.sync_copy(bt_h.at[pl.ds(BATCH, TAIL)], tailb)
        pltpu.sync_copy(tailb, obt_h.at[pl.ds(BATCH, TAIL)])


@jax.jit
def _er_update(x, y, t16, bx, by, bt):
    xf = x.reshape(R1)
    bxf = bx.reshape(TOT)
    mesh = plsc.VectorSubcoreMesh(core_axis_name="c", subcore_axis_name="s")
    run = pl.kernel(
        _body,
        out_type=(
            jax.ShapeDtypeStruct((TOT,), jnp.float32),
            jax.ShapeDtypeStruct((BUFFER_SIZE,), jnp.int32),
            jax.ShapeDtypeStruct((BUFFER_SIZE,), jnp.int32),
        ),
        mesh=mesh,
        scratch_types=[
            pltpu.VMEM_SHARED((16, 2, CHUNK), jnp.float32),
            pltpu.VMEM((BATCH,), jnp.int32),
            pltpu.VMEM((BUFFER_SIZE - BATCH,), jnp.int32),
            pltpu.VMEM((16,), jnp.int32),
            pltpu.SemaphoreType.DMA,
            pltpu.SemaphoreType.DMA,
        ],
    )
    obx, oby, obt = run(xf, y, t16, bxf, by, bt)
    return obx.reshape(bx.shape), oby, obt


def kernel(x, y, task_id, bx, by, bt):
    t16 = jnp.full((16,), task_id, dtype=jnp.int32)
    return _er_update(x, y, t16, bx, by, bt)
